# Initial kernel scaffold; baseline (speedup 1.0000x reference)
import jax
import jax.numpy as jnp
from jax.experimental import pallas as pl


def kernel(edges, W, bias):
    raise NotImplementedError("write your pallas kernel here")


# speedup vs baseline: 237.2504x; 237.2504x over previous
import functools

import jax
import jax.numpy as jnp
from jax import lax
from jax.experimental import pallas as pl
from jax.experimental.pallas import tpu as pltpu
from jax.experimental.pallas import tpu_sc as plsc

B = 1024
N = 100000
NPAD = 100096
K = 52
M = 26
NC = 2
NS = 16
NW = NC * NS
ROWS = B // NW
FLAT = ROWS * K
IDXW = 128
NIDX = FLAT // IDXW
CHUNK_OFFS = (0, 16, 32, 36)
A_IN_C1 = 10


@functools.cache
def _build_singlenet_sc():
    return functools.partial(
        pl.kernel,
        out_type=jax.ShapeDtypeStruct((B,), jnp.float32),
        mesh=plsc.VectorSubcoreMesh(core_axis_name="c", subcore_axis_name="s"),
        compiler_params=pltpu.CompilerParams(needs_layout_passes=False),
        scratch_types=[
            pltpu.VMEM((FLAT,), jnp.int32),
            pltpu.VMEM((FLAT,), jnp.float32),
            pltpu.VMEM((NPAD,), jnp.int32),
            pltpu.VMEM((ROWS,), jnp.float32),
            pltpu.VMEM((16,), jnp.float32),
            pltpu.SemaphoreType.DMA,
        ],
    )(_singlenet_sc)


def _singlenet_sc(edges_hbm, w_hbm, bias_hbm, out_hbm,
                  edges_v, wbuf, stamp, outv, bias_v, sem):
    wid = lax.axis_index("s") * NC + lax.axis_index("c")
    base = wid * FLAT

    pltpu.sync_copy(edges_hbm.at[pl.ds(base, FLAT)], edges_v)
    pltpu.sync_copy(bias_hbm, bias_v)

    copies = [
        pltpu.async_copy(
            w_hbm.at[edges_v.at[pl.ds(j * IDXW, IDXW)]],
            wbuf.at[pl.ds(j * IDXW, IDXW)],
            sem,
        )
        for j in range(NIDX)
    ]
    for c in copies:
        c.wait()

    lanes = lax.iota(jnp.int32, 16)
    ids = [lanes + 16 * c for c in range(4)]
    c1_a = lanes < A_IN_C1
    sign_pos = jnp.full((16,), 1.0, jnp.float32)
    sign_neg = jnp.full((16,), -1.0, jnp.float32)
    signs = [sign_pos, jnp.where(c1_a, 1.0, -1.0).astype(jnp.float32),
             sign_neg, sign_neg]
    zero = jnp.zeros((16,), jnp.float32)
    lane0 = lanes == 0

    for i in range(ROWS):
        o = i * K
        e = [edges_v[pl.ds(o + off, 16)] for off in CHUNK_OFFS]
        w = [wbuf[pl.ds(o + off, 16)] for off in CHUNK_OFFS]
        plsc.store_scatter(stamp, [e[0]], ids[0])
        plsc.store_scatter(stamp, [e[1]], ids[1], mask=c1_a)
        plsc.store_scatter(stamp, [e[1]], ids[1], mask=~c1_a)
        plsc.store_scatter(stamp, [e[2]], ids[2])
        plsc.store_scatter(stamp, [e[3]], ids[3])
        acc = zero
        for c in range(4):
            s = plsc.load_gather(stamp, [e[c]])
            acc = acc + jnp.where(s == ids[c], w[c] * signs[c], zero)
        tot = zero + jnp.sum(acc)
        plsc.store_scatter(outv, [lanes * 0 + i], tot, mask=lane0)

    for c in range(ROWS // 16):
        x = outv[pl.ds(c * 16, 16)] + bias_v[...]
        outv[pl.ds(c * 16, 16)] = 1.0 / (1.0 + jnp.exp(-x))

    pltpu.sync_copy(outv, out_hbm.at[pl.ds(wid * ROWS, ROWS)])


@jax.jit
def kernel(edges, W, bias):
    edges_flat = edges.astype(jnp.int32).reshape(-1)
    w_flat = W.reshape(-1).astype(jnp.float32)
    bias16 = jnp.broadcast_to(bias.astype(jnp.float32), (16,))
    pred = _build_singlenet_sc()(edges_flat, w_flat, bias16)
    return pred.reshape(B, 1)
